# Initial kernel scaffold; baseline (speedup 1.0000x reference)
import jax
import jax.numpy as jnp
from jax.experimental import pallas as pl


def kernel(x_movie, x_director, edge_index_movie_to_director, edge_index_director_to_movie, W_rel_m2d, b_rel_m2d, W_root_m2d, W_rel_d2m, b_rel_d2m, W_root_d2m):
    raise NotImplementedError("write your pallas kernel here")


# speedup vs baseline: 2.2785x; 2.2785x over previous
import functools

import jax
import jax.numpy as jnp
from jax import lax
from jax.experimental import pallas as pl
from jax.experimental.pallas import tpu as pltpu
from jax.experimental.pallas import tpu_sc as plsc

N = 100000
D = 128
E = 300000
EPAD = 303104
EPT = EPAD // 16
SEG = 2368
NSEG = EPT // SEG
GSEG = SEG // 16
R = 10000
NBINS = 10
NPASS = NBINS // 2
TRASH = R
RPT = R // 16
K = 128
RING = 32


def _tile_rows_loop(body, n64, tail):
    def f(k2, _):
        body(k2 * 64, 64)
        return 0
    lax.fori_loop(0, n64, f, 0)
    if tail:
        body(n64 * 64, tail)


def _sc_segment_sums(y_mov, y_dir, src1, dst1, src2, dst2):
    mesh = plsc.VectorSubcoreMesh(core_axis_name="c", subcore_axis_name="s")

    @functools.partial(
        pl.kernel,
        out_type=[
            jax.ShapeDtypeStruct((N, D), jnp.float32),
            jax.ShapeDtypeStruct((N, D), jnp.float32),
        ],
        mesh=mesh,
        compiler_params=pltpu.CompilerParams(use_tc_tiling_on_sc=False,
                                             needs_layout_passes=False),
        scratch_types=[
            pltpu.VMEM((SEG,), jnp.int32),
            pltpu.VMEM((SEG,), jnp.int32),
            pltpu.VMEM((RING, K), jnp.int32),
            pltpu.VMEM((RING, K), jnp.int32),
            pltpu.VMEM((K, D), jnp.float32),
            pltpu.VMEM((64, D), jnp.float32),
            pltpu.VMEM_SHARED((R + 8, D), jnp.float32),
            pltpu.SemaphoreType.DMA,
        ],
    )
    def body(y_mov_h, y_dir_h, src1_h, dst1_h, src2_h, dst2_h,
             agg_dir_h, agg_mov_h,
             esrc, edst, gidx, sidx, rowbuf, zblk, acc, sem):
        cid = lax.axis_index("c")
        sid = lax.axis_index("s")
        zero16 = jnp.zeros((16,), jnp.float32)
        iota16 = lax.iota(jnp.int32, 16)

        def zb(t, _):
            zblk[t >> 3, pl.ds((t & 7) * 16, 16)] = zero16
            return 0
        lax.fori_loop(0, 64 * 8, zb, 0)

        def do_chunk(c2, y_h):
            pltpu.async_copy(y_h.at[gidx.at[c2 & (RING - 1)]], rowbuf,
                             sem).wait()
            pltpu.sync_copy(rowbuf, acc.at[sidx.at[c2 & (RING - 1)]],
                            add=True)

        def relation(y_h, src_h, dst_h, out_h):
            r0 = sid * RPT

            def pass_body(p, _):
                base = (p * 2 + cid) * R

                def zf(off, nr):
                    pltpu.sync_copy(zblk.at[pl.ds(0, nr)],
                                    acc.at[pl.ds(r0 + off, nr)])
                _tile_rows_loop(zf, 9, 49)

                @pl.when(sid == 0)
                def _():
                    pltpu.sync_copy(zblk.at[pl.ds(0, 8)],
                                    acc.at[pl.ds(R, 8)])
                plsc.subcore_barrier()

                def seg_body(s, carry):
                    ptr, done = carry
                    off = sid * EPT + s * SEG
                    pltpu.sync_copy(src_h.at[pl.ds(off, SEG)], esrc)
                    pltpu.sync_copy(dst_h.at[pl.ds(off, SEG)], edst)

                    def fgrp(g, ptr):
                        dv = edst[pl.ds(g * 16, 16)]
                        m = (dv >= base) & (dv < base + R)
                        mi = m.astype(jnp.int32)
                        pos = ptr + plsc.cumsum(mi) - 1
                        row = lax.bitwise_and(
                            lax.shift_right_arithmetic(pos, 7), RING - 1)
                        col = lax.bitwise_and(pos, K - 1)
                        sv = esrc[pl.ds(g * 16, 16)]
                        plsc.store_scatter(gidx, [row, col], sv, mask=m)
                        plsc.store_scatter(sidx, [row, col], dv - base,
                                           mask=m)
                        return ptr + jnp.sum(mi)
                    ptr = lax.fori_loop(0, GSEG, fgrp, ptr)

                    ready = lax.shift_right_arithmetic(ptr, 7)

                    def ch(c2, _):
                        do_chunk(c2, y_h)
                        return 0
                    lax.fori_loop(done, ready, ch, 0)
                    return (ptr, ready)

                ptr, done = lax.fori_loop(0, NSEG, seg_body,
                                          (jnp.int32(0), jnp.int32(0)))

                nch = lax.shift_right_arithmetic(ptr + (K - 1), 7)
                p2 = nch * K

                def padb(t, _):
                    pos = ptr + t * 16 + iota16
                    pm = pos < p2
                    prow = lax.bitwise_and(
                        lax.shift_right_arithmetic(pos, 7), RING - 1)
                    pcol = lax.bitwise_and(pos, K - 1)
                    plsc.store_scatter(gidx, [prow, pcol],
                                       jnp.zeros((16,), jnp.int32), mask=pm)
                    plsc.store_scatter(sidx, [prow, pcol],
                                       jnp.full((16,), TRASH, jnp.int32),
                                       mask=pm)
                    return 0
                lax.fori_loop(0, K // 16, padb, 0)

                @pl.when(nch > done)
                def _():
                    do_chunk(done, y_h)
                plsc.subcore_barrier()

                def wb(off, nr):
                    pltpu.sync_copy(acc.at[pl.ds(r0 + off, nr)],
                                    out_h.at[pl.ds(base + r0 + off, nr)])
                _tile_rows_loop(wb, 9, 49)
                return 0

            lax.fori_loop(0, NPASS, pass_body, 0)
            plsc.subcore_barrier()

        relation(y_mov_h, src1_h, dst1_h, agg_dir_h)
        relation(y_dir_h, src2_h, dst2_h, agg_mov_h)

    return body(y_mov, y_dir, src1, dst1, src2, dst2)


BLK = 1000


def _mm_body(x_ref, w_ref, o_ref):
    o_ref[...] = jnp.dot(x_ref[...], w_ref[...],
                         preferred_element_type=jnp.float32)


def _tc_messages(x, w):
    return pl.pallas_call(
        _mm_body,
        grid=(N // BLK,),
        in_specs=[
            pl.BlockSpec((BLK, D), lambda i: (i, 0)),
            pl.BlockSpec((D, D), lambda i: (0, 0)),
        ],
        out_specs=pl.BlockSpec((BLK, D), lambda i: (i, 0)),
        out_shape=jax.ShapeDtypeStruct((N, D), jnp.float32),
    )(x, w)


def _fin_body(agg_ref, x_ref, w_ref, b_ref, o_ref):
    o_ref[...] = jnp.maximum(
        agg_ref[...]
        + jnp.dot(x_ref[...], w_ref[...], preferred_element_type=jnp.float32)
        + b_ref[...],
        0.0,
    )


def _tc_finish(agg, x, w_root, b):
    return pl.pallas_call(
        _fin_body,
        grid=(N // BLK,),
        in_specs=[
            pl.BlockSpec((BLK, D), lambda i: (i, 0)),
            pl.BlockSpec((BLK, D), lambda i: (i, 0)),
            pl.BlockSpec((D, D), lambda i: (0, 0)),
            pl.BlockSpec((1, D), lambda i: (0, 0)),
        ],
        out_specs=pl.BlockSpec((BLK, D), lambda i: (i, 0)),
        out_shape=jax.ShapeDtypeStruct((N, D), jnp.float32),
    )(agg, x, w_root, b)


def kernel(x_movie, x_director, edge_index_movie_to_director,
           edge_index_director_to_movie, W_rel_m2d, b_rel_m2d, W_root_m2d,
           W_rel_d2m, b_rel_d2m, W_root_d2m):
    y_mov = _tc_messages(x_movie, W_rel_m2d)
    y_dir = _tc_messages(x_director, W_rel_d2m)

    pad_src = jnp.zeros((EPAD - E,), jnp.int32)
    pad_dst = jnp.full((EPAD - E,), jnp.int32(1 << 30))
    src1 = jnp.concatenate([edge_index_movie_to_director[0], pad_src])
    dst1 = jnp.concatenate([edge_index_movie_to_director[1], pad_dst])
    src2 = jnp.concatenate([edge_index_director_to_movie[0], pad_src])
    dst2 = jnp.concatenate([edge_index_director_to_movie[1], pad_dst])

    agg_dir, agg_mov = _sc_segment_sums(y_mov, y_dir, src1, dst1, src2, dst2)

    out_director = _tc_finish(agg_dir, x_director, W_root_m2d,
                              b_rel_m2d.reshape(1, D))
    out_movie = _tc_finish(agg_mov, x_movie, W_root_d2m,
                           b_rel_d2m.reshape(1, D))
    return (out_movie, out_director)


# speedup vs baseline: 2.4142x; 1.0596x over previous
import functools

import jax
import jax.numpy as jnp
from jax import lax
from jax.experimental import pallas as pl
from jax.experimental.pallas import tpu as pltpu
from jax.experimental.pallas import tpu_sc as plsc

N = 100000
D = 128
E = 300000
EPAD = 303104
EPT = EPAD // 16
SEG = 1184
NSEG = EPT // SEG
GSEG = SEG // 16
R = 10000
NBINS = 10
NPASS = NBINS // 2
TRASH = R
RPT = R // 16
K = 128
RING = 16
ZR = 64


def _sc_segment_sums(y_mov, y_dir, src1, dst1, src2, dst2):
    mesh = plsc.VectorSubcoreMesh(core_axis_name="c", subcore_axis_name="s")

    @functools.partial(
        pl.kernel,
        out_type=[
            jax.ShapeDtypeStruct((N, D), jnp.float32),
            jax.ShapeDtypeStruct((N, D), jnp.float32),
        ],
        mesh=mesh,
        compiler_params=pltpu.CompilerParams(use_tc_tiling_on_sc=False,
                                             needs_layout_passes=False),
        scratch_types=[
            pltpu.VMEM((2, SEG), jnp.int32),
            pltpu.VMEM((2, SEG), jnp.int32),
            pltpu.VMEM((RING, K), jnp.int32),
            pltpu.VMEM((RING, K), jnp.int32),
            pltpu.VMEM((K, D), jnp.float32),
            pltpu.VMEM((K, D), jnp.float32),
            pltpu.VMEM((ZR, D), jnp.float32),
            pltpu.VMEM_SHARED((R + 8, D), jnp.float32),
            pltpu.SemaphoreType.DMA,
            pltpu.SemaphoreType.DMA,
            pltpu.SemaphoreType.DMA,
            pltpu.SemaphoreType.DMA,
        ],
    )
    def body(y_mov_h, y_dir_h, src1_h, dst1_h, src2_h, dst2_h,
             agg_dir_h, agg_mov_h,
             esrcb, edstb, gidx, sidx, rb0, rb1, zblk, acc,
             zsem, esem, gsem0, gsem1):
        cid = lax.axis_index("c")
        sid = lax.axis_index("s")
        zero16 = jnp.zeros((16,), jnp.float32)
        iota16 = lax.iota(jnp.int32, 16)

        def zb(t, _):
            zblk[t >> 3, pl.ds((t & 7) * 16, 16)] = zero16
            return 0
        lax.fori_loop(0, ZR * 8, zb, 0)

        def relation(y_h, src_h, dst_h, out_h):
            r0 = sid * RPT
            ebase = sid * EPT

            def pass_body(p, _):
                base = (p * 2 + cid) * R

                def zf_issue(k2, _):
                    pltpu.async_copy(zblk.at[pl.ds(0, ZR)],
                                     acc.at[pl.ds(r0 + k2 * ZR, ZR)], zsem)
                    return 0
                lax.fori_loop(0, 9, zf_issue, 0)
                pltpu.async_copy(zblk.at[pl.ds(0, 49)],
                                 acc.at[pl.ds(r0 + 576, 49)], zsem)

                @pl.when(sid == 0)
                def _():
                    pltpu.async_copy(zblk.at[pl.ds(0, 8)],
                                     acc.at[pl.ds(R, 8)], zsem)

                pltpu.async_copy(src_h.at[pl.ds(ebase, SEG)],
                                 esrcb.at[0], esem)
                pltpu.async_copy(dst_h.at[pl.ds(ebase, SEG)],
                                 edstb.at[0], esem)

                def zf_drain(k2, _):
                    pltpu.make_async_copy(
                        zblk.at[pl.ds(0, ZR)],
                        acc.at[pl.ds(r0 + k2 * ZR, ZR)], zsem).wait()
                    return 0
                lax.fori_loop(0, 9, zf_drain, 0)
                pltpu.make_async_copy(zblk.at[pl.ds(0, 49)],
                                      acc.at[pl.ds(r0 + 576, 49)],
                                      zsem).wait()

                @pl.when(sid == 0)
                def _():
                    pltpu.make_async_copy(zblk.at[pl.ds(0, 8)],
                                          acc.at[pl.ds(R, 8)], zsem).wait()
                plsc.subcore_barrier()

                def seg_body(s, carry):
                    ptr_v, done = carry
                    par = lax.bitwise_and(s, 1)
                    off = ebase + s * SEG
                    pltpu.make_async_copy(src_h.at[pl.ds(off, SEG)],
                                          esrcb.at[par], esem).wait()
                    pltpu.make_async_copy(dst_h.at[pl.ds(off, SEG)],
                                          edstb.at[par], esem).wait()

                    @pl.when(s + 1 < NSEG)
                    def _():
                        off2 = off + SEG
                        pltpu.async_copy(src_h.at[pl.ds(off2, SEG)],
                                         esrcb.at[1 - par], esem)
                        pltpu.async_copy(dst_h.at[pl.ds(off2, SEG)],
                                         edstb.at[1 - par], esem)

                    def fgrp(g, ptr_v):
                        dv = edstb[par, pl.ds(g * 16, 16)]
                        m = (dv >= base) & (dv < base + R)
                        pos = ptr_v + plsc.cumsum(m.astype(jnp.int32)) - 1
                        row = lax.bitwise_and(
                            lax.shift_right_arithmetic(pos, 7), RING - 1)
                        col = lax.bitwise_and(pos, K - 1)
                        sv = esrcb[par, pl.ds(g * 16, 16)]
                        plsc.store_scatter(gidx, [row, col], sv, mask=m)
                        plsc.store_scatter(sidx, [row, col], dv - base,
                                           mask=m)
                        return ptr_v + plsc.all_reduce_population_count(m)
                    ptr_v = lax.fori_loop(0, GSEG, fgrp, ptr_v)

                    ready = lax.shift_right_arithmetic(jnp.max(ptr_v), 7)

                    def emit(c2, _):
                        def work(rb, rb_o, gs, gs_o):
                            cr = lax.bitwise_and(c2, RING - 1)
                            cr1 = lax.bitwise_and(c2 + 1, RING - 1)

                            @pl.when(c2 == done)
                            def _():
                                pltpu.async_copy(y_h.at[gidx.at[cr]], rb,
                                                 gs)
                            pltpu.make_async_copy(y_h.at[gidx.at[cr]], rb,
                                                  gs).wait()

                            @pl.when(c2 + 1 < ready)
                            def _():
                                pltpu.async_copy(y_h.at[gidx.at[cr1]],
                                                 rb_o, gs_o)
                            pltpu.sync_copy(rb, acc.at[sidx.at[cr]],
                                            add=True)

                        @pl.when(lax.bitwise_and(c2, 1) == 0)
                        def _():
                            work(rb0, rb1, gsem0, gsem1)

                        @pl.when(lax.bitwise_and(c2, 1) == 1)
                        def _():
                            work(rb1, rb0, gsem1, gsem0)
                        return 0
                    lax.fori_loop(done, ready, emit, 0)
                    return (ptr_v, ready)

                init = (jnp.zeros((16,), jnp.int32), jnp.int32(0))
                ptr_v, done = lax.fori_loop(0, NSEG, seg_body, init)
                ptr = jnp.max(ptr_v)

                nch = lax.shift_right_arithmetic(ptr + (K - 1), 7)
                p2 = nch * K

                def padb(t, _):
                    pos = ptr + t * 16 + iota16
                    pm = pos < p2
                    prow = lax.bitwise_and(
                        lax.shift_right_arithmetic(pos, 7), RING - 1)
                    pcol = lax.bitwise_and(pos, K - 1)
                    plsc.store_scatter(gidx, [prow, pcol],
                                       jnp.zeros((16,), jnp.int32), mask=pm)
                    plsc.store_scatter(sidx, [prow, pcol],
                                       jnp.full((16,), TRASH, jnp.int32),
                                       mask=pm)
                    return 0
                lax.fori_loop(0, K // 16, padb, 0)

                @pl.when(nch > done)
                def _():
                    cr = lax.bitwise_and(done, RING - 1)
                    pltpu.async_copy(y_h.at[gidx.at[cr]], rb0, gsem0).wait()
                    pltpu.sync_copy(rb0, acc.at[sidx.at[cr]], add=True)
                plsc.subcore_barrier()

                def wb_issue(k2, _):
                    pltpu.async_copy(
                        acc.at[pl.ds(r0 + k2 * ZR, ZR)],
                        out_h.at[pl.ds(base + r0 + k2 * ZR, ZR)], zsem)
                    return 0
                lax.fori_loop(0, 9, wb_issue, 0)
                pltpu.async_copy(acc.at[pl.ds(r0 + 576, 49)],
                                 out_h.at[pl.ds(base + r0 + 576, 49)], zsem)

                def wb_drain(k2, _):
                    pltpu.make_async_copy(
                        acc.at[pl.ds(r0 + k2 * ZR, ZR)],
                        out_h.at[pl.ds(base + r0 + k2 * ZR, ZR)],
                        zsem).wait()
                    return 0
                lax.fori_loop(0, 9, wb_drain, 0)
                pltpu.make_async_copy(acc.at[pl.ds(r0 + 576, 49)],
                                      out_h.at[pl.ds(base + r0 + 576, 49)],
                                      zsem).wait()
                return 0

            lax.fori_loop(0, NPASS, pass_body, 0)
            plsc.subcore_barrier()

        relation(y_mov_h, src1_h, dst1_h, agg_dir_h)
        relation(y_dir_h, src2_h, dst2_h, agg_mov_h)

    return body(y_mov, y_dir, src1, dst1, src2, dst2)


BLK = 1000


def _mm_body(x_ref, w_ref, o_ref):
    o_ref[...] = jnp.dot(x_ref[...], w_ref[...],
                         preferred_element_type=jnp.float32)


def _tc_messages(x, w):
    return pl.pallas_call(
        _mm_body,
        grid=(N // BLK,),
        in_specs=[
            pl.BlockSpec((BLK, D), lambda i: (i, 0)),
            pl.BlockSpec((D, D), lambda i: (0, 0)),
        ],
        out_specs=pl.BlockSpec((BLK, D), lambda i: (i, 0)),
        out_shape=jax.ShapeDtypeStruct((N, D), jnp.float32),
    )(x, w)


def _fin_body(agg_ref, x_ref, w_ref, b_ref, o_ref):
    o_ref[...] = jnp.maximum(
        agg_ref[...]
        + jnp.dot(x_ref[...], w_ref[...], preferred_element_type=jnp.float32)
        + b_ref[...],
        0.0,
    )


def _tc_finish(agg, x, w_root, b):
    return pl.pallas_call(
        _fin_body,
        grid=(N // BLK,),
        in_specs=[
            pl.BlockSpec((BLK, D), lambda i: (i, 0)),
            pl.BlockSpec((BLK, D), lambda i: (i, 0)),
            pl.BlockSpec((D, D), lambda i: (0, 0)),
            pl.BlockSpec((1, D), lambda i: (0, 0)),
        ],
        out_specs=pl.BlockSpec((BLK, D), lambda i: (i, 0)),
        out_shape=jax.ShapeDtypeStruct((N, D), jnp.float32),
    )(agg, x, w_root, b)


def kernel(x_movie, x_director, edge_index_movie_to_director,
           edge_index_director_to_movie, W_rel_m2d, b_rel_m2d, W_root_m2d,
           W_rel_d2m, b_rel_d2m, W_root_d2m):
    y_mov = _tc_messages(x_movie, W_rel_m2d)
    y_dir = _tc_messages(x_director, W_rel_d2m)

    pad_src = jnp.zeros((EPAD - E,), jnp.int32)
    pad_dst = jnp.full((EPAD - E,), jnp.int32(1 << 30))
    src1 = jnp.concatenate([edge_index_movie_to_director[0], pad_src])
    dst1 = jnp.concatenate([edge_index_movie_to_director[1], pad_dst])
    src2 = jnp.concatenate([edge_index_director_to_movie[0], pad_src])
    dst2 = jnp.concatenate([edge_index_director_to_movie[1], pad_dst])

    agg_dir, agg_mov = _sc_segment_sums(y_mov, y_dir, src1, dst1, src2, dst2)

    out_director = _tc_finish(agg_dir, x_director, W_root_m2d,
                              b_rel_m2d.reshape(1, D))
    out_movie = _tc_finish(agg_mov, x_movie, W_root_d2m,
                           b_rel_d2m.reshape(1, D))
    return (out_movie, out_director)


# speedup vs baseline: 2.6230x; 1.0865x over previous
import functools

import jax
import jax.numpy as jnp
from jax import lax
from jax.experimental import pallas as pl
from jax.experimental.pallas import tpu as pltpu
from jax.experimental.pallas import tpu_sc as plsc

N = 100000
D = 128
E = 300000
EPAD = 303104
EPT = EPAD // 16
SEG = 1184
NSEG = EPT // SEG
GSEG = SEG // 16
R = 10000
NBINS = 10
NPASS = NBINS // 2
TRASH = R
RPT = R // 16
K = 128
RING = 16
ZR = 64


def _sc_segment_sums(y_mov, y_dir, src1, dst1, src2, dst2):
    mesh = plsc.VectorSubcoreMesh(core_axis_name="c", subcore_axis_name="s")

    @functools.partial(
        pl.kernel,
        out_type=[
            jax.ShapeDtypeStruct((N, D), jnp.float32),
            jax.ShapeDtypeStruct((N, D), jnp.float32),
        ],
        mesh=mesh,
        compiler_params=pltpu.CompilerParams(use_tc_tiling_on_sc=False,
                                             needs_layout_passes=False),
        scratch_types=[
            pltpu.VMEM((2, SEG), jnp.int32),
            pltpu.VMEM((2, SEG), jnp.int32),
            pltpu.VMEM((RING, K), jnp.int32),
            pltpu.VMEM((RING, K), jnp.int32),
            pltpu.VMEM((K, D), jnp.float32),
            pltpu.VMEM((K, D), jnp.float32),
            pltpu.VMEM((ZR, D), jnp.float32),
            pltpu.VMEM_SHARED((R + 8, D), jnp.float32),
            pltpu.SemaphoreType.DMA,
            pltpu.SemaphoreType.DMA,
            pltpu.SemaphoreType.DMA,
            pltpu.SemaphoreType.DMA,
        ],
    )
    def body(y_mov_h, y_dir_h, src1_h, dst1_h, src2_h, dst2_h,
             agg_dir_h, agg_mov_h,
             esrcb, edstb, gidx, sidx, rb0, rb1, zblk, acc,
             zsem, esem, gsem0, gsem1):
        cid = lax.axis_index("c")
        sid = lax.axis_index("s")
        zero16 = jnp.zeros((16,), jnp.float32)
        iota16 = lax.iota(jnp.int32, 16)

        def zb(t, _):
            zblk[t >> 3, pl.ds((t & 7) * 16, 16)] = zero16
            return 0
        lax.fori_loop(0, ZR * 8, zb, 0)

        def relation(y_h, src_h, dst_h, out_h):
            r0 = sid * RPT
            ebase = sid * EPT

            def pass_body(p, _):
                base = (p * 2 + cid) * R

                def zf_issue(k2, _):
                    pltpu.async_copy(zblk.at[pl.ds(0, ZR)],
                                     acc.at[pl.ds(r0 + k2 * ZR, ZR)], zsem)
                    return 0
                lax.fori_loop(0, 9, zf_issue, 0)
                pltpu.async_copy(zblk.at[pl.ds(0, 49)],
                                 acc.at[pl.ds(r0 + 576, 49)], zsem)

                @pl.when(sid == 0)
                def _():
                    pltpu.async_copy(zblk.at[pl.ds(0, 8)],
                                     acc.at[pl.ds(R, 8)], zsem)

                pltpu.async_copy(src_h.at[pl.ds(ebase, SEG)],
                                 esrcb.at[0], esem)
                pltpu.async_copy(dst_h.at[pl.ds(ebase, SEG)],
                                 edstb.at[0], esem)

                def zf_drain(k2, _):
                    pltpu.make_async_copy(
                        zblk.at[pl.ds(0, ZR)],
                        acc.at[pl.ds(r0 + k2 * ZR, ZR)], zsem).wait()
                    return 0
                lax.fori_loop(0, 9, zf_drain, 0)
                pltpu.make_async_copy(zblk.at[pl.ds(0, 49)],
                                      acc.at[pl.ds(r0 + 576, 49)],
                                      zsem).wait()

                @pl.when(sid == 0)
                def _():
                    pltpu.make_async_copy(zblk.at[pl.ds(0, 8)],
                                          acc.at[pl.ds(R, 8)], zsem).wait()
                plsc.subcore_barrier()

                def seg_body(s, carry):
                    ptr_v, done = carry
                    par = lax.bitwise_and(s, 1)
                    off = ebase + s * SEG
                    pltpu.make_async_copy(src_h.at[pl.ds(off, SEG)],
                                          esrcb.at[par], esem).wait()
                    pltpu.make_async_copy(dst_h.at[pl.ds(off, SEG)],
                                          edstb.at[par], esem).wait()

                    @pl.when(s + 1 < NSEG)
                    def _():
                        off2 = off + SEG
                        pltpu.async_copy(src_h.at[pl.ds(off2, SEG)],
                                         esrcb.at[1 - par], esem)
                        pltpu.async_copy(dst_h.at[pl.ds(off2, SEG)],
                                         edstb.at[1 - par], esem)

                    def fgrp(g, ptr_v):
                        dv = edstb[par, pl.ds(g * 16, 16)]
                        m = (dv >= base) & (dv < base + R)
                        pos = ptr_v + plsc.cumsum(m.astype(jnp.int32)) - 1
                        row = lax.bitwise_and(
                            lax.shift_right_arithmetic(pos, 7), RING - 1)
                        col = lax.bitwise_and(pos, K - 1)
                        sv = esrcb[par, pl.ds(g * 16, 16)]
                        plsc.store_scatter(gidx, [row, col], sv, mask=m)
                        plsc.store_scatter(sidx, [row, col], dv - base,
                                           mask=m)
                        return ptr_v + plsc.all_reduce_population_count(m)
                    ptr_v = plsc.parallel_loop(0, GSEG, unroll=4,
                                               carry=ptr_v)(fgrp)

                    ready = lax.shift_right_arithmetic(jnp.max(ptr_v), 7)

                    def emit(c2, _):
                        def work(rb, rb_o, gs, gs_o):
                            cr = lax.bitwise_and(c2, RING - 1)
                            cr1 = lax.bitwise_and(c2 + 1, RING - 1)

                            @pl.when(c2 == done)
                            def _():
                                pltpu.async_copy(y_h.at[gidx.at[cr]], rb,
                                                 gs)
                            pltpu.make_async_copy(y_h.at[gidx.at[cr]], rb,
                                                  gs).wait()

                            @pl.when(c2 + 1 < ready)
                            def _():
                                pltpu.async_copy(y_h.at[gidx.at[cr1]],
                                                 rb_o, gs_o)
                            pltpu.sync_copy(rb, acc.at[sidx.at[cr]],
                                            add=True)

                        @pl.when(lax.bitwise_and(c2, 1) == 0)
                        def _():
                            work(rb0, rb1, gsem0, gsem1)

                        @pl.when(lax.bitwise_and(c2, 1) == 1)
                        def _():
                            work(rb1, rb0, gsem1, gsem0)
                        return 0
                    lax.fori_loop(done, ready, emit, 0)
                    return (ptr_v, ready)

                init = (jnp.zeros((16,), jnp.int32), jnp.int32(0))
                ptr_v, done = lax.fori_loop(0, NSEG, seg_body, init)
                ptr = jnp.max(ptr_v)

                nch = lax.shift_right_arithmetic(ptr + (K - 1), 7)
                p2 = nch * K

                def padb(t, _):
                    pos = ptr + t * 16 + iota16
                    pm = pos < p2
                    prow = lax.bitwise_and(
                        lax.shift_right_arithmetic(pos, 7), RING - 1)
                    pcol = lax.bitwise_and(pos, K - 1)
                    plsc.store_scatter(gidx, [prow, pcol],
                                       jnp.zeros((16,), jnp.int32), mask=pm)
                    plsc.store_scatter(sidx, [prow, pcol],
                                       jnp.full((16,), TRASH, jnp.int32),
                                       mask=pm)
                    return 0
                lax.fori_loop(0, K // 16, padb, 0)

                @pl.when(nch > done)
                def _():
                    cr = lax.bitwise_and(done, RING - 1)
                    pltpu.async_copy(y_h.at[gidx.at[cr]], rb0, gsem0).wait()
                    pltpu.sync_copy(rb0, acc.at[sidx.at[cr]], add=True)
                plsc.subcore_barrier()

                def wb_issue(k2, _):
                    pltpu.async_copy(
                        acc.at[pl.ds(r0 + k2 * ZR, ZR)],
                        out_h.at[pl.ds(base + r0 + k2 * ZR, ZR)], zsem)
                    return 0
                lax.fori_loop(0, 9, wb_issue, 0)
                pltpu.async_copy(acc.at[pl.ds(r0 + 576, 49)],
                                 out_h.at[pl.ds(base + r0 + 576, 49)], zsem)

                def wb_drain(k2, _):
                    pltpu.make_async_copy(
                        acc.at[pl.ds(r0 + k2 * ZR, ZR)],
                        out_h.at[pl.ds(base + r0 + k2 * ZR, ZR)],
                        zsem).wait()
                    return 0
                lax.fori_loop(0, 9, wb_drain, 0)
                pltpu.make_async_copy(acc.at[pl.ds(r0 + 576, 49)],
                                      out_h.at[pl.ds(base + r0 + 576, 49)],
                                      zsem).wait()
                return 0

            lax.fori_loop(0, NPASS, pass_body, 0)
            plsc.subcore_barrier()

        relation(y_mov_h, src1_h, dst1_h, agg_dir_h)
        relation(y_dir_h, src2_h, dst2_h, agg_mov_h)

    return body(y_mov, y_dir, src1, dst1, src2, dst2)


BLK = 1000


def _mm_body(x_ref, w_ref, o_ref):
    o_ref[...] = jnp.dot(x_ref[...], w_ref[...],
                         preferred_element_type=jnp.float32)


def _tc_messages(x, w):
    return pl.pallas_call(
        _mm_body,
        grid=(N // BLK,),
        in_specs=[
            pl.BlockSpec((BLK, D), lambda i: (i, 0)),
            pl.BlockSpec((D, D), lambda i: (0, 0)),
        ],
        out_specs=pl.BlockSpec((BLK, D), lambda i: (i, 0)),
        out_shape=jax.ShapeDtypeStruct((N, D), jnp.float32),
    )(x, w)


def _fin_body(agg_ref, x_ref, w_ref, b_ref, o_ref):
    o_ref[...] = jnp.maximum(
        agg_ref[...]
        + jnp.dot(x_ref[...], w_ref[...], preferred_element_type=jnp.float32)
        + b_ref[...],
        0.0,
    )


def _tc_finish(agg, x, w_root, b):
    return pl.pallas_call(
        _fin_body,
        grid=(N // BLK,),
        in_specs=[
            pl.BlockSpec((BLK, D), lambda i: (i, 0)),
            pl.BlockSpec((BLK, D), lambda i: (i, 0)),
            pl.BlockSpec((D, D), lambda i: (0, 0)),
            pl.BlockSpec((1, D), lambda i: (0, 0)),
        ],
        out_specs=pl.BlockSpec((BLK, D), lambda i: (i, 0)),
        out_shape=jax.ShapeDtypeStruct((N, D), jnp.float32),
    )(agg, x, w_root, b)


def kernel(x_movie, x_director, edge_index_movie_to_director,
           edge_index_director_to_movie, W_rel_m2d, b_rel_m2d, W_root_m2d,
           W_rel_d2m, b_rel_d2m, W_root_d2m):
    y_mov = _tc_messages(x_movie, W_rel_m2d)
    y_dir = _tc_messages(x_director, W_rel_d2m)

    pad_src = jnp.zeros((EPAD - E,), jnp.int32)
    pad_dst = jnp.full((EPAD - E,), jnp.int32(1 << 30))
    src1 = jnp.concatenate([edge_index_movie_to_director[0], pad_src])
    dst1 = jnp.concatenate([edge_index_movie_to_director[1], pad_dst])
    src2 = jnp.concatenate([edge_index_director_to_movie[0], pad_src])
    dst2 = jnp.concatenate([edge_index_director_to_movie[1], pad_dst])

    agg_dir, agg_mov = _sc_segment_sums(y_mov, y_dir, src1, dst1, src2, dst2)

    out_director = _tc_finish(agg_dir, x_director, W_root_m2d,
                              b_rel_m2d.reshape(1, D))
    out_movie = _tc_finish(agg_mov, x_movie, W_root_d2m,
                           b_rel_d2m.reshape(1, D))
    return (out_movie, out_director)


# speedup vs baseline: 2.6255x; 1.0010x over previous
import functools

import jax
import jax.numpy as jnp
from jax import lax
from jax.experimental import pallas as pl
from jax.experimental.pallas import tpu as pltpu
from jax.experimental.pallas import tpu_sc as plsc

N = 100000
D = 128
E = 300000
EPAD = 303104
EPT = EPAD // 16
SEG = 1184
NSEG = EPT // SEG
GSEG = SEG // 16
R = 10000
NBINS = 10
NPASS = NBINS // 2
TRASH = R
RPT = R // 16
K = 128
RING = 16
ZR = 64


def _sc_segment_sums(y_mov, y_dir, src1, dst1, src2, dst2):
    mesh = plsc.VectorSubcoreMesh(core_axis_name="c", subcore_axis_name="s")

    @functools.partial(
        pl.kernel,
        out_type=[
            jax.ShapeDtypeStruct((N, D), jnp.float32),
            jax.ShapeDtypeStruct((N, D), jnp.float32),
        ],
        mesh=mesh,
        compiler_params=pltpu.CompilerParams(use_tc_tiling_on_sc=False,
                                             needs_layout_passes=False),
        scratch_types=[
            pltpu.VMEM((2, SEG), jnp.int32),
            pltpu.VMEM((2, SEG), jnp.int32),
            pltpu.VMEM((RING, K), jnp.int32),
            pltpu.VMEM((RING, K), jnp.int32),
            pltpu.VMEM((K, D), jnp.float32),
            pltpu.VMEM((K, D), jnp.float32),
            pltpu.VMEM((ZR, D), jnp.float32),
            pltpu.VMEM_SHARED((R + 8, D), jnp.float32),
            pltpu.SemaphoreType.DMA,
            pltpu.SemaphoreType.DMA,
            pltpu.SemaphoreType.DMA,
            pltpu.SemaphoreType.DMA,
        ],
    )
    def body(y_mov_h, y_dir_h, src1_h, dst1_h, src2_h, dst2_h,
             agg_dir_h, agg_mov_h,
             esrcb, edstb, gidx, sidx, rb0, rb1, zblk, acc,
             zsem, esem, gsem0, gsem1):
        cid = lax.axis_index("c")
        sid = lax.axis_index("s")
        zero16 = jnp.zeros((16,), jnp.float32)
        iota16 = lax.iota(jnp.int32, 16)

        def zb(t, _):
            zblk[t >> 3, pl.ds((t & 7) * 16, 16)] = zero16
            return 0
        lax.fori_loop(0, ZR * 8, zb, 0)

        def relation(y_h, src_h, dst_h, out_h):
            r0 = sid * RPT
            ebase = sid * EPT

            def pass_body(p, _):
                base = (p * 2 + cid) * R

                def zf_issue(k2, _):
                    pltpu.async_copy(zblk.at[pl.ds(0, ZR)],
                                     acc.at[pl.ds(r0 + k2 * ZR, ZR)], zsem)
                    return 0
                lax.fori_loop(0, 9, zf_issue, 0)
                pltpu.async_copy(zblk.at[pl.ds(0, 49)],
                                 acc.at[pl.ds(r0 + 576, 49)], zsem)

                @pl.when(sid == 0)
                def _():
                    pltpu.async_copy(zblk.at[pl.ds(0, 8)],
                                     acc.at[pl.ds(R, 8)], zsem)

                pltpu.async_copy(src_h.at[pl.ds(ebase, SEG)],
                                 esrcb.at[0], esem)
                pltpu.async_copy(dst_h.at[pl.ds(ebase, SEG)],
                                 edstb.at[0], esem)

                def zf_drain(k2, _):
                    pltpu.make_async_copy(
                        zblk.at[pl.ds(0, ZR)],
                        acc.at[pl.ds(r0 + k2 * ZR, ZR)], zsem).wait()
                    return 0
                lax.fori_loop(0, 9, zf_drain, 0)
                pltpu.make_async_copy(zblk.at[pl.ds(0, 49)],
                                      acc.at[pl.ds(r0 + 576, 49)],
                                      zsem).wait()

                @pl.when(sid == 0)
                def _():
                    pltpu.make_async_copy(zblk.at[pl.ds(0, 8)],
                                          acc.at[pl.ds(R, 8)], zsem).wait()
                plsc.subcore_barrier()

                def seg_body(s, carry):
                    ptr_v, done = carry
                    par = lax.bitwise_and(s, 1)
                    off = ebase + s * SEG
                    pltpu.make_async_copy(src_h.at[pl.ds(off, SEG)],
                                          esrcb.at[par], esem).wait()
                    pltpu.make_async_copy(dst_h.at[pl.ds(off, SEG)],
                                          edstb.at[par], esem).wait()

                    @pl.when(s + 1 < NSEG)
                    def _():
                        off2 = off + SEG
                        pltpu.async_copy(src_h.at[pl.ds(off2, SEG)],
                                         esrcb.at[1 - par], esem)
                        pltpu.async_copy(dst_h.at[pl.ds(off2, SEG)],
                                         edstb.at[1 - par], esem)

                    def fgrp(g, ptr_v):
                        dv = edstb[par, pl.ds(g * 16, 16)]
                        m = (dv >= base) & (dv < base + R)
                        pos = ptr_v + plsc.cumsum(m.astype(jnp.int32)) - 1
                        row = lax.bitwise_and(
                            lax.shift_right_arithmetic(pos, 7), RING - 1)
                        col = lax.bitwise_and(pos, K - 1)
                        sv = esrcb[par, pl.ds(g * 16, 16)]
                        plsc.store_scatter(gidx, [row, col], sv, mask=m)
                        plsc.store_scatter(sidx, [row, col], dv - base,
                                           mask=m)
                        return ptr_v + plsc.all_reduce_population_count(m)
                    ptr_v = plsc.parallel_loop(0, GSEG, unroll=8,
                                               carry=ptr_v)(fgrp)

                    ready = lax.shift_right_arithmetic(jnp.max(ptr_v), 7)

                    def emit(c2, _):
                        def work(rb, rb_o, gs, gs_o):
                            cr = lax.bitwise_and(c2, RING - 1)
                            cr1 = lax.bitwise_and(c2 + 1, RING - 1)

                            @pl.when(c2 == done)
                            def _():
                                pltpu.async_copy(y_h.at[gidx.at[cr]], rb,
                                                 gs)
                            pltpu.make_async_copy(y_h.at[gidx.at[cr]], rb,
                                                  gs).wait()

                            @pl.when(c2 + 1 < ready)
                            def _():
                                pltpu.async_copy(y_h.at[gidx.at[cr1]],
                                                 rb_o, gs_o)
                            pltpu.sync_copy(rb, acc.at[sidx.at[cr]],
                                            add=True)

                        @pl.when(lax.bitwise_and(c2, 1) == 0)
                        def _():
                            work(rb0, rb1, gsem0, gsem1)

                        @pl.when(lax.bitwise_and(c2, 1) == 1)
                        def _():
                            work(rb1, rb0, gsem1, gsem0)
                        return 0
                    lax.fori_loop(done, ready, emit, 0)
                    return (ptr_v, ready)

                init = (jnp.zeros((16,), jnp.int32), jnp.int32(0))
                ptr_v, done = lax.fori_loop(0, NSEG, seg_body, init)
                ptr = jnp.max(ptr_v)

                nch = lax.shift_right_arithmetic(ptr + (K - 1), 7)
                p2 = nch * K

                def padb(t, _):
                    pos = ptr + t * 16 + iota16
                    pm = pos < p2
                    prow = lax.bitwise_and(
                        lax.shift_right_arithmetic(pos, 7), RING - 1)
                    pcol = lax.bitwise_and(pos, K - 1)
                    plsc.store_scatter(gidx, [prow, pcol],
                                       jnp.zeros((16,), jnp.int32), mask=pm)
                    plsc.store_scatter(sidx, [prow, pcol],
                                       jnp.full((16,), TRASH, jnp.int32),
                                       mask=pm)
                    return 0
                lax.fori_loop(0, K // 16, padb, 0)

                @pl.when(nch > done)
                def _():
                    cr = lax.bitwise_and(done, RING - 1)
                    pltpu.async_copy(y_h.at[gidx.at[cr]], rb0, gsem0).wait()
                    pltpu.sync_copy(rb0, acc.at[sidx.at[cr]], add=True)
                plsc.subcore_barrier()

                def wb_issue(k2, _):
                    pltpu.async_copy(
                        acc.at[pl.ds(r0 + k2 * ZR, ZR)],
                        out_h.at[pl.ds(base + r0 + k2 * ZR, ZR)], zsem)
                    return 0
                lax.fori_loop(0, 9, wb_issue, 0)
                pltpu.async_copy(acc.at[pl.ds(r0 + 576, 49)],
                                 out_h.at[pl.ds(base + r0 + 576, 49)], zsem)

                def wb_drain(k2, _):
                    pltpu.make_async_copy(
                        acc.at[pl.ds(r0 + k2 * ZR, ZR)],
                        out_h.at[pl.ds(base + r0 + k2 * ZR, ZR)],
                        zsem).wait()
                    return 0
                lax.fori_loop(0, 9, wb_drain, 0)
                pltpu.make_async_copy(acc.at[pl.ds(r0 + 576, 49)],
                                      out_h.at[pl.ds(base + r0 + 576, 49)],
                                      zsem).wait()
                return 0

            lax.fori_loop(0, NPASS, pass_body, 0)
            plsc.subcore_barrier()

        relation(y_mov_h, src1_h, dst1_h, agg_dir_h)
        relation(y_dir_h, src2_h, dst2_h, agg_mov_h)

    return body(y_mov, y_dir, src1, dst1, src2, dst2)


BLK = 1000


def _mm_body(x_ref, w_ref, o_ref):
    o_ref[...] = jnp.dot(x_ref[...], w_ref[...],
                         preferred_element_type=jnp.float32)


def _tc_messages(x, w):
    return pl.pallas_call(
        _mm_body,
        grid=(N // BLK,),
        in_specs=[
            pl.BlockSpec((BLK, D), lambda i: (i, 0)),
            pl.BlockSpec((D, D), lambda i: (0, 0)),
        ],
        out_specs=pl.BlockSpec((BLK, D), lambda i: (i, 0)),
        out_shape=jax.ShapeDtypeStruct((N, D), jnp.float32),
    )(x, w)


def _fin_body(agg_ref, x_ref, w_ref, b_ref, o_ref):
    o_ref[...] = jnp.maximum(
        agg_ref[...]
        + jnp.dot(x_ref[...], w_ref[...], preferred_element_type=jnp.float32)
        + b_ref[...],
        0.0,
    )


def _tc_finish(agg, x, w_root, b):
    return pl.pallas_call(
        _fin_body,
        grid=(N // BLK,),
        in_specs=[
            pl.BlockSpec((BLK, D), lambda i: (i, 0)),
            pl.BlockSpec((BLK, D), lambda i: (i, 0)),
            pl.BlockSpec((D, D), lambda i: (0, 0)),
            pl.BlockSpec((1, D), lambda i: (0, 0)),
        ],
        out_specs=pl.BlockSpec((BLK, D), lambda i: (i, 0)),
        out_shape=jax.ShapeDtypeStruct((N, D), jnp.float32),
    )(agg, x, w_root, b)


def kernel(x_movie, x_director, edge_index_movie_to_director,
           edge_index_director_to_movie, W_rel_m2d, b_rel_m2d, W_root_m2d,
           W_rel_d2m, b_rel_d2m, W_root_d2m):
    y_mov = _tc_messages(x_movie, W_rel_m2d)
    y_dir = _tc_messages(x_director, W_rel_d2m)

    pad_src = jnp.zeros((EPAD - E,), jnp.int32)
    pad_dst = jnp.full((EPAD - E,), jnp.int32(1 << 30))
    src1 = jnp.concatenate([edge_index_movie_to_director[0], pad_src])
    dst1 = jnp.concatenate([edge_index_movie_to_director[1], pad_dst])
    src2 = jnp.concatenate([edge_index_director_to_movie[0], pad_src])
    dst2 = jnp.concatenate([edge_index_director_to_movie[1], pad_dst])

    agg_dir, agg_mov = _sc_segment_sums(y_mov, y_dir, src1, dst1, src2, dst2)

    out_director = _tc_finish(agg_dir, x_director, W_root_m2d,
                              b_rel_m2d.reshape(1, D))
    out_movie = _tc_finish(agg_mov, x_movie, W_root_d2m,
                           b_rel_d2m.reshape(1, D))
    return (out_movie, out_director)
